# Initial kernel scaffold; baseline (speedup 1.0000x reference)
import jax
import jax.numpy as jnp
from jax.experimental import pallas as pl


def kernel(xyzt, gate_W, gate_b, P, tb1, tb2, W0, g1, g2, W1, W2, Wf, bf):
    raise NotImplementedError("write your pallas kernel here")


# speedup vs baseline: 2.6924x; 2.6924x over previous
import functools

import jax
import jax.numpy as jnp
from jax import lax
from jax.experimental import pallas as pl
from jax.scipy.linalg import block_diag

E = 8
TOPK = 2
HID = 16
PROJ = 8

_F32 = jnp.float32
_BF16 = jnp.bfloat16


def _moe_block_kernel(xt_ref, aux_ref, p_ref, w0_ref, w1_ref, w2_ref,
                      wf_ref, bf_ref, o_ref):
    xt = xt_ref[...]
    tb = xt.shape[1]
    x0 = xt[0:1, :]
    x1 = xt[1:2, :]
    x2 = xt[2:3, :]
    t = xt[3:4, :]

    rho = jnp.sqrt(x0 * x0 + x1 * x1 + x2 * x2)
    z = jnp.clip(x2 / rho, -1.0, 1.0)
    at = jnp.arctan2(jnp.concatenate([x1, jnp.sqrt(1.0 - z * z)], axis=0),
                     jnp.concatenate([x0, z], axis=0))

    sph = jnp.concatenate(
        [rho, at, jnp.zeros((5, tb), _F32)], axis=0)
    proj = jnp.dot(p_ref[...], sph, preferred_element_type=_F32)

    aux = aux_ref[...]
    b1 = aux[:, 5:6]
    b2 = aux[:, 6:7]
    ct = jnp.cos(t + b1)
    st = jnp.sin(t + b2)
    cs = jnp.concatenate([ct, st], axis=0)
    h0 = jnp.concatenate([proj, cs, jax.nn.silu(cs)], axis=0)

    def geglu(u):
        return u[:128, :] * jax.nn.gelu(u[128:, :])

    r128 = lax.broadcasted_iota(jnp.int32, (128, 8), 0)
    c128 = lax.broadcasted_iota(jnp.int32, (128, 8), 1)
    g = (r128 // HID == c128).astype(_F32)
    gt = g.T
    r32 = lax.broadcasted_iota(jnp.int32, (32, 8), 0)
    c32 = lax.broadcasted_iota(jnp.int32, (32, 8), 1)
    k4 = (r32 // 4 == c32).astype(_F32)
    r4 = lax.broadcasted_iota(jnp.int32, (4, 32), 0)
    c4 = lax.broadcasted_iota(jnp.int32, (4, 32), 1)
    s = (c4 % 4 == r4).astype(_F32)
    hx = lax.Precision.HIGHEST

    def rms_scale(h):
        ss = jnp.dot(gt, h * h, preferred_element_type=_F32, precision=hx)
        inv = 1.0 / (jnp.sqrt(ss) * (HID ** -0.5) + 1e-8)
        return h * jnp.dot(g, inv, preferred_element_type=_F32, precision=hx)

    h = geglu(jnp.dot(w0_ref[...], h0, preferred_element_type=_F32))
    h = geglu(jnp.dot(w1_ref[...], rms_scale(h), preferred_element_type=_F32))
    h = geglu(jnp.dot(w2_ref[...], rms_scale(h), preferred_element_type=_F32))
    y = jnp.dot(wf_ref[...], h, preferred_element_type=_F32) + bf_ref[:, 0:1]

    gwt = aux[:, 0:4]
    gb = aux[:, 4:5]
    logits = jnp.dot(gwt, xt, preferred_element_type=_F32) + gb
    def cumsum8(v):
        zeros = jnp.zeros_like(v)
        for k in (1, 2, 4):
            v = v + jnp.concatenate([zeros[:k, :], v[:-k, :]], axis=0)
        return v

    m1 = jnp.max(logits, axis=0, keepdims=True)
    eq1 = (logits == m1).astype(_F32)
    c1 = cumsum8(eq1)
    first = eq1 * (c1 == 1.0).astype(_F32)
    masked = logits - 1e30 * first
    m2 = jnp.max(masked, axis=0, keepdims=True)
    eq2 = (masked == m2).astype(_F32)
    c2 = cumsum8(eq2)
    sec = eq2 * (c2 == 1.0).astype(_F32)
    e2 = jnp.exp(m2 - m1)
    w_hi = 1.0 / (1.0 + e2)
    w_lo = 1.0 - w_hi
    wdense = first * w_hi + sec * w_lo

    wrep = jnp.dot(k4, wdense, preferred_element_type=_F32, precision=hx)
    o_ref[...] = jnp.dot(s, wrep * y, preferred_element_type=_F32, precision=hx)


@functools.partial(jax.jit, static_argnames=("interpret", "tb"))
def _run(xt, aux, p_t, w0t, w1t, w2t, wft, bfp, *, interpret=False, tb=2048):
    t_tot = xt.shape[1]
    grid = (t_tot // tb,)
    full = lambda a: pl.BlockSpec(a.shape, lambda i: (0, 0))
    return pl.pallas_call(
        _moe_block_kernel,
        grid=grid,
        in_specs=[
            pl.BlockSpec((4, tb), lambda i: (0, i)),
            full(aux), full(p_t),
            full(w0t), full(w1t), full(w2t), full(wft), full(bfp),
        ],
        out_specs=pl.BlockSpec((4, tb), lambda i: (0, i)),
        out_shape=jax.ShapeDtypeStruct((4, t_tot), _F32),
        interpret=interpret,
    )(xt, aux, p_t, w0t, w1t, w2t, wft, bfp)


def _prep(gate_W, gate_b, P, tb1, tb2, W0, g1, g2, W1, W2, Wf, bf):
    aux = jnp.concatenate([
        gate_W.astype(_F32).T,
        gate_b.reshape(E, 1).astype(_F32),
        tb1.reshape(E, 1).astype(_F32),
        tb2.reshape(E, 1).astype(_F32),
        jnp.zeros((E, 1), _F32),
    ], axis=1)

    p_t = jnp.concatenate(
        [jnp.transpose(P, (0, 2, 1)).reshape(E * PROJ, 3).astype(_F32),
         jnp.zeros((E * PROJ, 5), _F32)], axis=1)

    def stack_w0(wh):
        parts = [block_diag(*[wh[e, :PROJ, :] for e in range(E)])]
        for f in range(4):
            parts.append(
                block_diag(*[wh[e, PROJ + f:PROJ + f + 1, :] for e in range(E)]))
        return jnp.concatenate(parts, axis=0)

    w0t = jnp.concatenate(
        [stack_w0(W0[:, :, :HID]), stack_w0(W0[:, :, HID:])], axis=1).T

    def bd_pair(w):
        return jnp.concatenate(
            [block_diag(*[w[e, :, :HID] for e in range(E)]),
             block_diag(*[w[e, :, HID:] for e in range(E)])], axis=1).T

    w1t = bd_pair(g1[:, :, None] * W1)
    w2t = bd_pair(g2[:, :, None] * W2)
    wft = block_diag(*[Wf[e] for e in range(E)]).T
    bfp = jnp.concatenate(
        [bf.reshape(E * 4, 1), jnp.zeros((E * 4, 7), _F32)], axis=1)

    return aux, p_t, w0t, w1t, w2t, wft, bfp


def kernel(xyzt, gate_W, gate_b, P, tb1, tb2, W0, g1, g2, W1, W2, Wf, bf):
    B, N, D = xyzt.shape
    xt = xyzt.reshape(B * N, D).astype(_F32).T
    packed = _prep(gate_W, gate_b, P, tb1, tb2, W0, g1, g2, W1, W2, Wf, bf)
    out = _run(xt, *packed)
    return out.T.reshape(B, N, 4)


# speedup vs baseline: 4.3440x; 1.6134x over previous
import functools

import jax
import jax.numpy as jnp
from jax import lax
from jax.experimental import pallas as pl


E = 8
TOPK = 2
HID = 16
PROJ = 8

_F32 = jnp.float32
_BF16 = jnp.bfloat16


def _moe_block_kernel(xt_ref, aux_ref, p_ref, w0_ref, w1_ref, w2_ref,
                      wf_ref, bf_ref, o_ref):
    xt = xt_ref[...]
    tb = xt.shape[1]
    x0 = xt[0:1, :]
    x1 = xt[1:2, :]
    x2 = xt[2:3, :]
    t = xt[3:4, :]

    rho = jnp.sqrt(x0 * x0 + x1 * x1 + x2 * x2)
    z = jnp.clip(x2 / rho, -1.0, 1.0)
    at = jnp.arctan2(jnp.concatenate([x1, jnp.sqrt(1.0 - z * z)], axis=0),
                     jnp.concatenate([x0, z], axis=0))

    sph = jnp.concatenate(
        [rho, at, jnp.zeros((5, tb), _F32)], axis=0)
    proj = jnp.dot(p_ref[...], sph, preferred_element_type=_F32)

    aux = aux_ref[...]
    b1 = aux[:, 5:6]
    b2 = aux[:, 6:7]
    ct = jnp.cos(t + b1)
    st = jnp.sin(t + b2)
    cs = jnp.concatenate([ct, st], axis=0)
    h0 = jnp.concatenate([proj, cs, jax.nn.silu(cs)], axis=0)

    def geglu(u):
        return u[:128, :] * jax.nn.gelu(u[128:, :])

    def rms_scale(h):
        ss = jnp.sum((h * h).reshape(E, HID, tb), axis=1)
        inv = 1.0 / (jnp.sqrt(ss) * (HID ** -0.5) + 1e-8)
        bc = jnp.broadcast_to(inv[:, None, :], (E, HID, tb))
        return h * bc.reshape(E * HID, tb)

    h = geglu(jnp.dot(w0_ref[...], h0, preferred_element_type=_F32))
    h = geglu(jnp.dot(w1_ref[...], rms_scale(h), preferred_element_type=_F32))
    h = geglu(jnp.dot(w2_ref[...], rms_scale(h), preferred_element_type=_F32))
    y = jnp.dot(wf_ref[...], h, preferred_element_type=_F32) + bf_ref[:, 0:1]

    gwt = aux[:, 0:4]
    gb = aux[:, 4:5]
    logits = jnp.dot(gwt, xt, preferred_element_type=_F32) + gb
    def cumsum8(v):
        zeros = jnp.zeros_like(v)
        for k in (1, 2, 4):
            v = v + jnp.concatenate([zeros[:k, :], v[:-k, :]], axis=0)
        return v

    m1 = jnp.max(logits, axis=0, keepdims=True)
    eq1 = (logits == m1).astype(_F32)
    c1 = cumsum8(eq1)
    first = eq1 * (c1 == 1.0).astype(_F32)
    masked = logits - 1e30 * first
    m2 = jnp.max(masked, axis=0, keepdims=True)
    eq2 = (masked == m2).astype(_F32)
    c2 = cumsum8(eq2)
    sec = eq2 * (c2 == 1.0).astype(_F32)
    e2 = jnp.exp(m2 - m1)
    w_hi = 1.0 / (1.0 + e2)
    w_lo = 1.0 - w_hi
    wdense = first * w_hi + sec * w_lo

    wrep = jnp.broadcast_to(wdense[:, None, :], (E, 4, tb)).reshape(E * 4, tb)
    o_ref[...] = jnp.sum((wrep * y).reshape(E, 4, tb), axis=0)


@functools.partial(jax.jit, static_argnames=("interpret", "tb"))
def _run(xt, aux, p_t, w0t, w1t, w2t, wft, bfp, *, interpret=False, tb=2048):
    t_tot = xt.shape[1]
    grid = (t_tot // tb,)
    full = lambda a: pl.BlockSpec(a.shape, lambda i: (0, 0))
    return pl.pallas_call(
        _moe_block_kernel,
        grid=grid,
        in_specs=[
            pl.BlockSpec((4, tb), lambda i: (0, i)),
            full(aux), full(p_t),
            full(w0t), full(w1t), full(w2t), full(wft), full(bfp),
        ],
        out_specs=pl.BlockSpec((4, tb), lambda i: (0, i)),
        out_shape=jax.ShapeDtypeStruct((4, t_tot), _F32),
        interpret=interpret,
    )(xt, aux, p_t, w0t, w1t, w2t, wft, bfp)


def _bdiag(a):
    e, m, n = a.shape
    w = (e + 1) * n
    a = jnp.pad(a, ((0, 0), (0, 0), (0, w - n)))
    a = a.reshape(e, m * w)
    a = jnp.pad(a, ((0, 0), (0, n)))
    a = a.reshape(-1)[: e * m * w].reshape(e * m, w)
    return a[:, : e * n]


def _prep(gate_W, gate_b, P, tb1, tb2, W0, g1, g2, W1, W2, Wf, bf):
    aux = jnp.concatenate([
        gate_W.astype(_F32).T,
        gate_b.reshape(E, 1).astype(_F32),
        tb1.reshape(E, 1).astype(_F32),
        tb2.reshape(E, 1).astype(_F32),
        jnp.zeros((E, 1), _F32),
    ], axis=1)

    p_t = jnp.concatenate(
        [jnp.transpose(P, (0, 2, 1)).reshape(E * PROJ, 3).astype(_F32),
         jnp.zeros((E * PROJ, 5), _F32)], axis=1)

    def stack_w0(wh):
        parts = [_bdiag(wh[:, :PROJ, :])]
        for f in range(4):
            parts.append(_bdiag(wh[:, PROJ + f:PROJ + f + 1, :]))
        return jnp.concatenate(parts, axis=0)

    w0t = jnp.concatenate(
        [stack_w0(W0[:, :, :HID]), stack_w0(W0[:, :, HID:])], axis=1).T

    def bd_pair(w):
        return jnp.concatenate(
            [_bdiag(w[:, :, :HID]), _bdiag(w[:, :, HID:])], axis=1).T

    w1t = bd_pair(g1[:, :, None] * W1)
    w2t = bd_pair(g2[:, :, None] * W2)
    wft = _bdiag(Wf).T
    bfp = jnp.concatenate(
        [bf.reshape(E * 4, 1), jnp.zeros((E * 4, 7), _F32)], axis=1)

    return aux, p_t, w0t, w1t, w2t, wft, bfp


def kernel(xyzt, gate_W, gate_b, P, tb1, tb2, W0, g1, g2, W1, W2, Wf, bf):
    B, N, D = xyzt.shape
    xt = xyzt.reshape(B * N, D).astype(_F32).T
    packed = _prep(gate_W, gate_b, P, tb1, tb2, W0, g1, g2, W1, W2, Wf, bf)
    out = _run(xt, *packed)
    return out.T.reshape(B, N, 4)
